# baseline (speedup 1.0000x reference)
import jax
import jax.numpy as jnp
from jax.experimental import pallas as pl
from jax.experimental.pallas import tpu as pltpu

_T = 20
_V = 32
_X = _T * _V
_G = 128
_NG = _X // _G
_BB = 64
_EPS = 1e-5


def _prelu(x, a):
    return jnp.where(x >= 0.0, x, a * x)


def _c1x1(x, w, b2):
    bf = jnp.bfloat16
    wb = jnp.broadcast_to(w.astype(bf)[None], (x.shape[0],) + w.shape)
    y = jnp.einsum('boc,bcx->box', wb, x.astype(bf),
                   preferred_element_type=jnp.float32)
    return y + b2[0][None, :, None]


def _gcn(g, m):
    b = g.shape[0]
    gb = g.astype(jnp.bfloat16)
    mb = m.astype(jnp.bfloat16)
    parts = [jnp.einsum('box,bxy->boy', gb[:, :, i * _G:(i + 1) * _G],
                        jnp.broadcast_to(mb[i][None], (b, _G, _G)),
                        preferred_element_type=jnp.float32)
             for i in range(_NG)]
    return jnp.concatenate(parts, axis=2)


def _tconv(y1, twf, tb2):
    ypad = jnp.pad(y1, ((0, 0), (0, 0), (_V, _V)))
    ycat = jnp.concatenate(
        [ypad[:, :, k * _V:k * _V + _X] for k in range(3)], axis=1)
    bf = jnp.bfloat16
    twb = jnp.broadcast_to(twf.astype(bf)[None], (y1.shape[0],) + twf.shape)
    y = jnp.einsum('boi,bix->box', twb, ycat.astype(bf),
                   preferred_element_type=jnp.float32)
    return y + tb2[0][None, :, None]


def _coeffs(part, r0, r1, g, b, count):
    s = jnp.sum(part, axis=0)
    mean = s[r0] / count
    var = s[r1] / count - mean * mean
    sc = g / jnp.sqrt(var + _EPS)
    return sc, b - mean * sc


def _stats4(y0, res):
    return jnp.concatenate([
        jnp.sum(y0, axis=(0, 2))[None, :],
        jnp.sum(y0 * y0, axis=(0, 2))[None, :],
        jnp.sum(res, axis=(0, 2))[None, :],
        jnp.sum(res * res, axis=(0, 2))[None, :],
    ], axis=0)


def _stats_body(x_ref, m_ref, gw_ref, gb_ref, rw_ref, rb_ref, acc_ref):
    x = x_ref[...]
    y0 = _gcn(_c1x1(x, gw_ref[...], gb_ref[...]), m_ref[...])
    res = _c1x1(x, rw_ref[...], rb_ref[...])
    acc_ref[0] = _stats4(y0, res)


def _mid_body(count, x_ref, m_ref, gw_ref, gb_ref, sp_ref, bn1_ref, a1_ref,
              twf_ref, tb_ref, y2_ref, acc_ref):
    sc0, sh0 = _coeffs(sp_ref[...], 0, 1, bn1_ref[0], bn1_ref[1], count)
    x = x_ref[...]
    y0 = _gcn(_c1x1(x, gw_ref[...], gb_ref[...]), m_ref[...])
    y1 = _prelu(y0 * sc0[None, :, None] + sh0[None, :, None], a1_ref[0, 0])
    y2 = _tconv(y1, twf_ref[...], tb_ref[...])
    y2_ref[...] = y2
    acc_ref[0] = jnp.concatenate([
        jnp.sum(y2, axis=(0, 2))[None, :],
        jnp.sum(y2 * y2, axis=(0, 2))[None, :],
    ], axis=0)


def _combine(count, y2_ref, x_ref, rwp_ref, rbp_ref, tp_ref, sp_ref,
             bn4_ref, a2_ref):
    sc2, sh2 = _coeffs(tp_ref[...], 0, 1, bn4_ref[0], bn4_ref[1], count)
    scr, shr = _coeffs(sp_ref[...], 2, 3, bn4_ref[2], bn4_ref[3], count)
    res = _c1x1(x_ref[...], rwp_ref[...], rbp_ref[...])
    t = (y2_ref[...] * sc2[None, :, None] + sh2[None, :, None]
         + res * scr[None, :, None] + shr[None, :, None])
    return _prelu(t, a2_ref[0, 0])


def _fused_body(count, y2_ref, x_ref, rwp_ref, rbp_ref, tp_ref, sp_ref,
                bn4_ref, a2_ref, m_ref, gwn_ref, gbn_ref, rwn_ref, rbn_ref,
                out_ref, accn_ref):
    out = _combine(count, y2_ref, x_ref, rwp_ref, rbp_ref, tp_ref, sp_ref,
                   bn4_ref, a2_ref)
    out_ref[...] = out
    y0n = _gcn(_c1x1(out, gwn_ref[...], gbn_ref[...]), m_ref[...])
    resn = _c1x1(out, rwn_ref[...], rbn_ref[...])
    accn_ref[0] = _stats4(y0n, resn)


def _final_body(count, y2_ref, x_ref, rwp_ref, rbp_ref, tp_ref, sp_ref,
                bn4_ref, a2_ref, out_ref):
    out_ref[...] = _combine(count, y2_ref, x_ref, rwp_ref, rbp_ref, tp_ref,
                            sp_ref, bn4_ref, a2_ref)


def _full(arr):
    nd = arr.ndim
    return pl.BlockSpec(arr.shape, lambda i, _nd=nd: (0,) * _nd)


def _bspec(c):
    return pl.BlockSpec((_BB, c, _X), lambda i: (i, 0, 0))


_PAR = pltpu.CompilerParams(dimension_semantics=("parallel",))


def _prep(p):
    f32 = jnp.float32
    return {
        'gw': p['gw'], 'rw': p['rw'],
        'gb': p['gb'][None, :], 'rb': p['rb'][None, :],
        'tb': p['tb'][None, :],
        'twf': jnp.concatenate([p['tw'][:, :, k, 0] for k in range(3)],
                               axis=1),
        'bn1': jnp.stack([p['g1'], p['b1']]),
        'bn4': jnp.stack([p['g2'], p['b2'], p['rg'], p['rbb']]),
        'a1': jnp.asarray(p['a1'], f32).reshape(1, 1),
        'a2': jnp.asarray(p['a2'], f32).reshape(1, 1),
    }


def kernel(v, a, params):
    n = v.shape[0]
    t, vv = a.shape[0], a.shape[1]
    f32 = jnp.float32
    x = v.reshape(n, v.shape[1], _X)
    nsteps = n // _BB
    grid = (nsteps,)
    count = float(n * _X)

    m4 = a[:, :, None, :] * jnp.eye(t, dtype=a.dtype)[:, None, :, None]
    mfull = m4.reshape(_X, _X)
    m = jnp.stack([mfull[i * _G:(i + 1) * _G, i * _G:(i + 1) * _G]
                   for i in range(_NG)])

    pp = {k: _prep(bp) for k, bp in params.items()}
    order = ['e1', 'e2', 'd1', 'd2']
    couts = {k: params[k]['gw'].shape[0] for k in order}

    def acc_shape(rows, c):
        return jax.ShapeDtypeStruct((nsteps, rows, c), f32)

    def acc_bspec(rows, c):
        return pl.BlockSpec((1, rows, c), lambda i: (i, 0, 0))

    q = pp['e1']
    sp = pl.pallas_call(
        _stats_body, grid=grid,
        in_specs=[_bspec(x.shape[1])] + [_full(z) for z in
                  (m, q['gw'], q['gb'], q['rw'], q['rb'])],
        out_specs=acc_bspec(4, couts['e1']),
        out_shape=acc_shape(4, couts['e1']),
        compiler_params=_PAR,
    )(x, m, q['gw'], q['gb'], q['rw'], q['rb'])

    outs = {}
    for bi, name in enumerate(order):
        q = pp[name]
        cin, cout = x.shape[1], couts[name]

        def mid(cnt):
            return lambda *refs: _mid_body(cnt, *refs)

        y2, tp = pl.pallas_call(
            mid(count), grid=grid,
            in_specs=[_bspec(cin)] + [_full(z) for z in
                      (m, q['gw'], q['gb'], sp, q['bn1'], q['a1'],
                       q['twf'], q['tb'])],
            out_specs=[_bspec(cout), acc_bspec(2, cout)],
            out_shape=[jax.ShapeDtypeStruct((n, cout, _X), f32),
                       acc_shape(2, cout)],
            compiler_params=_PAR,
        )(x, m, q['gw'], q['gb'], sp, q['bn1'], q['a1'], q['twf'], q['tb'])

        if bi < 3:
            nxt = pp[order[bi + 1]]
            cn = couts[order[bi + 1]]

            def fused(cnt):
                return lambda *refs: _fused_body(cnt, *refs)

            out, spn = pl.pallas_call(
                fused(count), grid=grid,
                in_specs=[_bspec(cout), _bspec(cin)] + [_full(z) for z in
                          (q['rw'], q['rb'], tp, sp, q['bn4'], q['a2'],
                           m, nxt['gw'], nxt['gb'], nxt['rw'], nxt['rb'])],
                out_specs=[_bspec(cout), acc_bspec(4, cn)],
                out_shape=[jax.ShapeDtypeStruct((n, cout, _X), f32),
                           acc_shape(4, cn)],
                compiler_params=_PAR,
            )(y2, x, q['rw'], q['rb'], tp, sp, q['bn4'], q['a2'],
              m, nxt['gw'], nxt['gb'], nxt['rw'], nxt['rb'])
            sp = spn
        else:
            def fin(cnt):
                return lambda *refs: _final_body(cnt, *refs)

            out = pl.pallas_call(
                fin(count), grid=grid,
                in_specs=[_bspec(cout), _bspec(cin)] + [_full(z) for z in
                          (q['rw'], q['rb'], tp, sp, q['bn4'], q['a2'])],
                out_specs=_bspec(cout),
                out_shape=jax.ShapeDtypeStruct((n, cout, _X), f32),
                compiler_params=_PAR,
            )(y2, x, q['rw'], q['rb'], tp, sp, q['bn4'], q['a2'])
        outs[name] = out
        x = out

    ev = outs['e2'].reshape(n, couts['e2'], t, vv)
    dv = outs['d2'].reshape(n, couts['d2'], t, vv)
    return (ev, dv)


# speedup vs baseline: 1.1239x; 1.1239x over previous
import jax
import jax.numpy as jnp
from jax.experimental import pallas as pl
from jax.experimental.pallas import tpu as pltpu

_T = 20
_V = 32
_X = _T * _V
_G = 128
_NG = _X // _G
_BB = 64
_EPS = 1e-5


def _prelu(x, a):
    return jnp.where(x >= 0.0, x, a * x)


def _c1x1(x, w, b2):
    wb = jnp.broadcast_to(w[None], (x.shape[0],) + w.shape)
    y = jnp.einsum('boc,bcx->box', wb, x, preferred_element_type=jnp.float32)
    return y + b2[0][None, :, None]


def _gcn(g, m):
    b = g.shape[0]
    parts = [jnp.einsum('box,bxy->boy', g[:, :, i * _G:(i + 1) * _G],
                        jnp.broadcast_to(m[i][None], (b, _G, _G)),
                        preferred_element_type=jnp.float32)
             for i in range(_NG)]
    return jnp.concatenate(parts, axis=2)


def _tconv(y1, twf, tb2):
    ypad = jnp.pad(y1, ((0, 0), (0, 0), (_V, _V)))
    ycat = jnp.concatenate(
        [ypad[:, :, k * _V:k * _V + _X] for k in range(3)], axis=1)
    twb = jnp.broadcast_to(twf[None], (y1.shape[0],) + twf.shape)
    y = jnp.einsum('boi,bix->box', twb, ycat, preferred_element_type=jnp.float32)
    return y + tb2[0][None, :, None]


def _coeffs(part, r0, r1, g, b, count):
    s = jnp.sum(part, axis=0)
    mean = s[r0] / count
    var = s[r1] / count - mean * mean
    sc = g / jnp.sqrt(var + _EPS)
    return sc, b - mean * sc


def _stats4(y0, res):
    return jnp.concatenate([
        jnp.sum(y0, axis=(0, 2))[None, :],
        jnp.sum(y0 * y0, axis=(0, 2))[None, :],
        jnp.sum(res, axis=(0, 2))[None, :],
        jnp.sum(res * res, axis=(0, 2))[None, :],
    ], axis=0)


def _stats_body(x_ref, m_ref, gw_ref, gb_ref, rw_ref, rb_ref, acc_ref):
    x = x_ref[...]
    y0 = _gcn(_c1x1(x, gw_ref[...], gb_ref[...]), m_ref[...])
    res = _c1x1(x, rw_ref[...], rb_ref[...])
    acc_ref[0] = _stats4(y0, res)


def _mid_body(count, x_ref, m_ref, gw_ref, gb_ref, sp_ref, bn1_ref, a1_ref,
              twf_ref, tb_ref, y2_ref, acc_ref):
    sc0, sh0 = _coeffs(sp_ref[...], 0, 1, bn1_ref[0], bn1_ref[1], count)
    x = x_ref[...]
    y0 = _gcn(_c1x1(x, gw_ref[...], gb_ref[...]), m_ref[...])
    y1 = _prelu(y0 * sc0[None, :, None] + sh0[None, :, None], a1_ref[0, 0])
    y2 = _tconv(y1, twf_ref[...], tb_ref[...])
    y2_ref[...] = y2
    acc_ref[0] = jnp.concatenate([
        jnp.sum(y2, axis=(0, 2))[None, :],
        jnp.sum(y2 * y2, axis=(0, 2))[None, :],
    ], axis=0)


def _combine(count, y2_ref, x_ref, rwp_ref, rbp_ref, tp_ref, sp_ref,
             bn4_ref, a2_ref):
    sc2, sh2 = _coeffs(tp_ref[...], 0, 1, bn4_ref[0], bn4_ref[1], count)
    scr, shr = _coeffs(sp_ref[...], 2, 3, bn4_ref[2], bn4_ref[3], count)
    res = _c1x1(x_ref[...], rwp_ref[...], rbp_ref[...])
    t = (y2_ref[...] * sc2[None, :, None] + sh2[None, :, None]
         + res * scr[None, :, None] + shr[None, :, None])
    return _prelu(t, a2_ref[0, 0])


def _fused_body(count, y2_ref, x_ref, rwp_ref, rbp_ref, tp_ref, sp_ref,
                bn4_ref, a2_ref, m_ref, gwn_ref, gbn_ref, rwn_ref, rbn_ref,
                out_ref, accn_ref):
    out = _combine(count, y2_ref, x_ref, rwp_ref, rbp_ref, tp_ref, sp_ref,
                   bn4_ref, a2_ref)
    out_ref[...] = out
    y0n = _gcn(_c1x1(out, gwn_ref[...], gbn_ref[...]), m_ref[...])
    resn = _c1x1(out, rwn_ref[...], rbn_ref[...])
    accn_ref[0] = _stats4(y0n, resn)


def _final_body(count, y2_ref, x_ref, rwp_ref, rbp_ref, tp_ref, sp_ref,
                bn4_ref, a2_ref, out_ref):
    out_ref[...] = _combine(count, y2_ref, x_ref, rwp_ref, rbp_ref, tp_ref,
                            sp_ref, bn4_ref, a2_ref)


def _full(arr):
    nd = arr.ndim
    return pl.BlockSpec(arr.shape, lambda i, _nd=nd: (0,) * _nd)


def _bspec(c, bb=_BB):
    return pl.BlockSpec((bb, c, _X), lambda i: (i, 0, 0))


_PAR = pltpu.CompilerParams(dimension_semantics=("parallel",))


def _prep(p):
    f32 = jnp.float32
    return {
        'gw': p['gw'], 'rw': p['rw'],
        'gb': p['gb'][None, :], 'rb': p['rb'][None, :],
        'tb': p['tb'][None, :],
        'twf': jnp.concatenate([p['tw'][:, :, k, 0] for k in range(3)],
                               axis=1),
        'bn1': jnp.stack([p['g1'], p['b1']]),
        'bn4': jnp.stack([p['g2'], p['b2'], p['rg'], p['rbb']]),
        'a1': jnp.asarray(p['a1'], f32).reshape(1, 1),
        'a2': jnp.asarray(p['a2'], f32).reshape(1, 1),
    }


def kernel(v, a, params):
    n = v.shape[0]
    t, vv = a.shape[0], a.shape[1]
    f32 = jnp.float32
    x = v.reshape(n, v.shape[1], _X)
    count = float(n * _X)

    m4 = a[:, :, None, :] * jnp.eye(t, dtype=a.dtype)[:, None, :, None]
    mfull = m4.reshape(_X, _X)
    m = jnp.stack([mfull[i * _G:(i + 1) * _G, i * _G:(i + 1) * _G]
                   for i in range(_NG)])

    pp = {k: _prep(bp) for k, bp in params.items()}
    order = ['e1', 'e2', 'd1', 'd2']
    couts = {k: params[k]['gw'].shape[0] for k in order}

    def acc_shape(rows, c, bb):
        return jax.ShapeDtypeStruct((n // bb, rows, c), f32)

    def acc_bspec(rows, c):
        return pl.BlockSpec((1, rows, c), lambda i: (i, 0, 0))

    q = pp['e1']
    bb = 128
    sp = pl.pallas_call(
        _stats_body, grid=(n // bb,),
        in_specs=[_bspec(x.shape[1], bb)] + [_full(z) for z in
                  (m, q['gw'], q['gb'], q['rw'], q['rb'])],
        out_specs=acc_bspec(4, couts['e1']),
        out_shape=acc_shape(4, couts['e1'], bb),
        compiler_params=_PAR,
    )(x, m, q['gw'], q['gb'], q['rw'], q['rb'])

    outs = {}
    for bi, name in enumerate(order):
        q = pp[name]
        cin, cout = x.shape[1], couts[name]

        def mid(cnt):
            return lambda *refs: _mid_body(cnt, *refs)

        bb = 64 if name == 'e2' else 128
        y2, tp = pl.pallas_call(
            mid(count), grid=(n // bb,),
            in_specs=[_bspec(cin, bb)] + [_full(z) for z in
                      (m, q['gw'], q['gb'], sp, q['bn1'], q['a1'],
                       q['twf'], q['tb'])],
            out_specs=[_bspec(cout, bb), acc_bspec(2, cout)],
            out_shape=[jax.ShapeDtypeStruct((n, cout, _X), f32),
                       acc_shape(2, cout, bb)],
            compiler_params=_PAR,
        )(x, m, q['gw'], q['gb'], sp, q['bn1'], q['a1'], q['twf'], q['tb'])

        if bi < 3:
            nxt = pp[order[bi + 1]]
            cn = couts[order[bi + 1]]

            def fused(cnt):
                return lambda *refs: _fused_body(cnt, *refs)

            bb = 64 if name == 'e2' else 128
            out, spn = pl.pallas_call(
                fused(count), grid=(n // bb,),
                in_specs=[_bspec(cout, bb), _bspec(cin, bb)] +
                         [_full(z) for z in
                          (q['rw'], q['rb'], tp, sp, q['bn4'], q['a2'],
                           m, nxt['gw'], nxt['gb'], nxt['rw'], nxt['rb'])],
                out_specs=[_bspec(cout, bb), acc_bspec(4, cn)],
                out_shape=[jax.ShapeDtypeStruct((n, cout, _X), f32),
                           acc_shape(4, cn, bb)],
                compiler_params=_PAR,
            )(y2, x, q['rw'], q['rb'], tp, sp, q['bn4'], q['a2'],
              m, nxt['gw'], nxt['gb'], nxt['rw'], nxt['rb'])
            sp = spn
        else:
            def fin(cnt):
                return lambda *refs: _final_body(cnt, *refs)

            bb = 128
            out = pl.pallas_call(
                fin(count), grid=(n // bb,),
                in_specs=[_bspec(cout, bb), _bspec(cin, bb)] +
                         [_full(z) for z in
                          (q['rw'], q['rb'], tp, sp, q['bn4'], q['a2'])],
                out_specs=_bspec(cout, bb),
                out_shape=jax.ShapeDtypeStruct((n, cout, _X), f32),
                compiler_params=_PAR,
            )(y2, x, q['rw'], q['rb'], tp, sp, q['bn4'], q['a2'])
        outs[name] = out
        x = out

    ev = outs['e2'].reshape(n, couts['e2'], t, vv)
    dv = outs['d2'].reshape(n, couts['d2'], t, vv)
    return (ev, dv)
